# Initial kernel scaffold; baseline (speedup 1.0000x reference)
import jax
import jax.numpy as jnp
from jax.experimental import pallas as pl


def kernel(x, Wq_adv, bq_adv, keys_adv, values_adv, Wq_val, bq_val, keys_val, values_val):
    raise NotImplementedError("write your pallas kernel here")


# speedup vs baseline: 3.6494x; 3.6494x over previous
import functools
import jax
import jax.numpy as jnp
from jax.experimental import pallas as pl
from jax.experimental.pallas import tpu as pltpu
from jax.experimental.pallas import tpu_sc as plsc

B = 16384
HIDDEN = 256
NUM_ACTIONS = 18
HEADS = 4
K_DIM = 256
HALF = K_DIM // 2
N_KEYS = 1024
KNN = 64
N_VALUES = N_KEYS * N_KEYS

TILE = 256

_PAIRS = [(a, b) for a in range(KNN) for b in range(KNN // (a + 1))]
N_CAND = len(_PAIRS)
N_CPAD = 384

import numpy as _np
_A_MAT = _np.zeros((KNN, N_CPAD), _np.float32)
_B_MAT = _np.zeros((KNN, N_CPAD), _np.float32)
_PAD_MASK = _np.full((1, N_CPAD), -1e30, _np.float32)
for _c, (_a, _b) in enumerate(_PAIRS):
    _A_MAT[_a, _c] = 1.0
    _B_MAT[_b, _c] = 1.0
    _PAD_MASK[0, _c] = 0.0

_NEG = float('-inf')


def _extract_topk(s, k, n_sentinel):
    R, N = s.shape
    iota = jax.lax.broadcasted_iota(jnp.int32, (R, N), 1)
    oiota = jax.lax.broadcasted_iota(jnp.int32, (R, k), 1)

    def body(j, carry):
        cur, outv, outp = carry
        m = jnp.max(cur, axis=1, keepdims=True)
        hit = cur == m
        p = jnp.min(jnp.where(hit, iota, n_sentinel), axis=1, keepdims=True)
        cur = jnp.where(iota == p, _NEG, cur)
        sel = oiota == j
        outv = jnp.where(sel, m, outv)
        outp = jnp.where(sel, p, outp)
        return cur, outv, outp

    outv = jnp.full((R, k), _NEG, jnp.float32)
    outp = jnp.zeros((R, k), jnp.int32)
    _, outv, outp = jax.lax.fori_loop(0, k, body, (s, outv, outp))
    return outv, outp


def _extract_topk_payload_f(s, payload, k, n_sentinel):
    R, N = s.shape
    iota = jax.lax.broadcasted_iota(jnp.int32, (R, N), 1)
    oiota = jax.lax.broadcasted_iota(jnp.int32, (R, k), 1)
    big = jnp.float32(3e9)

    def body(j, carry):
        cur, outv, outi = carry
        m = jnp.max(cur, axis=1, keepdims=True)
        hit = cur == m
        p = jnp.min(jnp.where(hit, iota, n_sentinel), axis=1, keepdims=True)
        at_p = iota == p
        pay = jnp.min(jnp.where(at_p, payload, big), axis=1, keepdims=True)
        cur = jnp.where(at_p, _NEG, cur)
        sel = oiota == j
        outv = jnp.where(sel, m, outv)
        outi = jnp.where(sel, pay, outi)
        return cur, outv, outi

    outv = jnp.full((R, k), _NEG, jnp.float32)
    outi = jnp.zeros((R, k), jnp.float32)
    _, outv, outi = jax.lax.fori_loop(0, k, body, (s, outv, outi))
    return outv, outi


def _select_kernel(x_ref, wq_ref, bq_ref, k_adv_ref, k_val_ref,
                   ab_ref, mask_ref, w_ref, idx_ref):
    x = x_ref[...]
    q = jnp.dot(x, wq_ref[...], preferred_element_type=jnp.float32)
    q = q + bq_ref[0, :]
    ab = ab_ref[...]
    pad_mask = mask_ref[0, :]
    hi = jax.lax.Precision.HIGHEST

    for pkm in range(2):
        k_ref = k_adv_ref if pkm == 0 else k_val_ref
        for h in range(HEADS):
            base = pkm * HEADS * K_DIM + h * K_DIM
            q1 = q[:, base:base + HALF]
            q2 = q[:, base + HALF:base + K_DIM]
            s1 = jax.lax.dot_general(
                q1, k_ref[2 * h], (((1,), (1,)), ((), ())),
                preferred_element_type=jnp.float32)
            s2 = jax.lax.dot_general(
                q2, k_ref[2 * h + 1], (((1,), (1,)), ((), ())),
                preferred_element_type=jnp.float32)
            sc1, i1 = _extract_topk(s1, KNN, N_KEYS)
            sc2, i2 = _extract_topk(s2, KNN, N_KEYS)

            sc12 = jnp.concatenate([sc1, sc2], axis=1)
            ix12 = jnp.concatenate(
                [i1.astype(jnp.float32) * float(N_KEYS),
                 i2.astype(jnp.float32)], axis=1)
            cand_sc = jnp.dot(sc12, ab, precision=hi,
                              preferred_element_type=jnp.float32) + pad_mask
            cand_if = jnp.dot(ix12, ab, precision=hi,
                              preferred_element_type=jnp.float32)

            bsc, fidx_f = _extract_topk_payload_f(cand_sc, cand_if, KNN,
                                                  N_CPAD)
            fidx = (fidx_f + 0.5).astype(jnp.int32)
            e = jnp.exp(bsc - bsc[:, :1])
            w = e / jnp.sum(e, axis=1, keepdims=True)
            out = (pkm * HEADS + h) * KNN
            w_ref[:, out:out + KNN] = w
            idx_ref[:, out:out + KNN] = fidx


def kernel(x, Wq_adv, bq_adv, keys_adv, values_adv, Wq_val, bq_val,
           keys_val, values_val):
    wq = jnp.concatenate([Wq_adv, Wq_val], axis=1)
    bq = jnp.concatenate([bq_adv, bq_val])[None, :]
    k_adv = keys_adv.reshape(HEADS * 2, N_KEYS, HALF)
    k_val = keys_val.reshape(HEADS * 2, N_KEYS, HALF)
    ab = jnp.asarray(_np.concatenate([_A_MAT, _B_MAT], axis=0))
    pad_mask = jnp.asarray(_PAD_MASK)

    grid = (B // TILE,)
    w_all, idx_all = pl.pallas_call(
        _select_kernel,
        grid=grid,
        in_specs=[
            pl.BlockSpec((TILE, HIDDEN), lambda i: (i, 0)),
            pl.BlockSpec((HIDDEN, 2 * HEADS * K_DIM), lambda i: (0, 0)),
            pl.BlockSpec((1, 2 * HEADS * K_DIM), lambda i: (0, 0)),
            pl.BlockSpec((HEADS * 2, N_KEYS, HALF), lambda i: (0, 0, 0)),
            pl.BlockSpec((HEADS * 2, N_KEYS, HALF), lambda i: (0, 0, 0)),
            pl.BlockSpec((2 * KNN, N_CPAD), lambda i: (0, 0)),
            pl.BlockSpec((1, N_CPAD), lambda i: (0, 0)),
        ],
        out_specs=[
            pl.BlockSpec((TILE, 2 * HEADS * KNN), lambda i: (i, 0)),
            pl.BlockSpec((TILE, 2 * HEADS * KNN), lambda i: (i, 0)),
        ],
        out_shape=[
            jax.ShapeDtypeStruct((B, 2 * HEADS * KNN), jnp.float32),
            jax.ShapeDtypeStruct((B, 2 * HEADS * KNN), jnp.int32),
        ],
    )(x, wq, bq, k_adv, k_val, ab, pad_mask)

    NPK = HEADS * KNN
    idx_adv = idx_all[:, :NPK].reshape(B * NPK)
    idx_val = idx_all[:, NPK:].reshape(B * NPK)
    vals_adv_pad = jnp.pad(values_adv, ((0, 0), (0, 128 - NUM_ACTIONS)))
    g_adv = _sc_gather(vals_adv_pad, idx_adv)
    g_val = _sc_gather_1d(values_val.reshape(N_VALUES), idx_val)
    g_val = g_val.reshape(B * NPK, 1)
    iv_val = idx_all[:, NPK:]

    out = pl.pallas_call(
        _reduce_kernel,
        grid=(B // RTILE,),
        in_specs=[
            pl.BlockSpec((RTILE, 2 * NPK), lambda i: (i, 0)),
            pl.BlockSpec((RTILE * NPK, 128), lambda i: (i, 0)),
            pl.BlockSpec((RTILE * NPK, 1), lambda i: (i, 0)),
            pl.BlockSpec((RTILE, NPK), lambda i: (i, 0)),
        ],
        out_specs=pl.BlockSpec((RTILE, NUM_ACTIONS), lambda i: (i, 0)),
        out_shape=jax.ShapeDtypeStruct((B, NUM_ACTIONS), jnp.float32),
    )(w_all, g_adv, g_val, iv_val)
    return out


VPAD = 32
VQ = 64
RTILE = 64
GCHUNK = 512
GCHUNKV = 2048


def _sc_gather(table, idx):
    m = idx.shape[0]
    info = plsc.get_sparse_core_info()
    nw = info.num_cores * info.num_subcores
    per_w = m // nw
    n_chunks = per_w // GCHUNK
    mesh = plsc.VectorSubcoreMesh(core_axis_name="c", subcore_axis_name="s")

    @functools.partial(
        pl.kernel,
        out_type=jax.ShapeDtypeStruct((m, 128), table.dtype),
        mesh=mesh,
        scratch_types=[
            pltpu.VMEM((GCHUNK,), jnp.int32),
            pltpu.VMEM((GCHUNK, 128), jnp.float32),
            pltpu.SemaphoreType.DMA,
        ])
    def gather_kernel(tab_hbm, i_hbm, o_hbm, idx_v, rows_v, sem):
        wid = jax.lax.axis_index("s") * info.num_cores + jax.lax.axis_index("c")
        base = wid * per_w

        @pl.loop(0, n_chunks)
        def _(i):
            off = base + i * GCHUNK
            pltpu.sync_copy(i_hbm.at[pl.ds(off, GCHUNK)], idx_v)
            pltpu.async_copy(tab_hbm.at[idx_v], rows_v, sem).wait()
            pltpu.sync_copy(rows_v, o_hbm.at[pl.ds(off, GCHUNK)])

    return gather_kernel(table, idx)


def _sc_gather_spmem(table, idx):
    m = idx.shape[0]
    n16 = table.shape[0]
    info = plsc.get_sparse_core_info()
    nw = info.num_cores * info.num_subcores
    per_w = m // nw
    n_chunks = per_w // GCHUNK
    mesh = plsc.VectorSubcoreMesh(core_axis_name="c", subcore_axis_name="s")

    @functools.partial(
        pl.kernel,
        out_type=jax.ShapeDtypeStruct((m, VQ), jnp.float32),
        mesh=mesh,
        scratch_types=[
            pltpu.VMEM((GCHUNK,), jnp.int32),
            pltpu.VMEM((GCHUNK, VQ), jnp.float32),
            pltpu.VMEM_SHARED((n16, VQ), jnp.float32),
            pltpu.SemaphoreType.DMA,
        ])
    def gather_kernel(tab_hbm, i_hbm, o_hbm, idx_v, rows_v, tab_sh, sem):
        sid = jax.lax.axis_index("s")

        @pl.when(sid == 0)
        def _():
            pltpu.sync_copy(tab_hbm, tab_sh)

        plsc.subcore_barrier()
        wid = sid * info.num_cores + jax.lax.axis_index("c")
        base = wid * per_w

        @pl.loop(0, n_chunks)
        def _(i):
            off = base + i * GCHUNK
            pltpu.sync_copy(i_hbm.at[pl.ds(off, GCHUNK)], idx_v)
            pltpu.async_copy(tab_sh.at[idx_v], rows_v, sem).wait()
            pltpu.sync_copy(rows_v, o_hbm.at[pl.ds(off, GCHUNK)])

    return gather_kernel(table, idx)


def _sc_gather_1d(table, idx):
    m = idx.shape[0]
    info = plsc.get_sparse_core_info()
    nw = info.num_cores * info.num_subcores
    per_w = m // nw
    n_chunks = per_w // GCHUNKV
    mesh = plsc.VectorSubcoreMesh(core_axis_name="c", subcore_axis_name="s")

    @functools.partial(
        pl.kernel,
        out_type=jax.ShapeDtypeStruct((m,), jnp.float32),
        mesh=mesh,
        scratch_types=[
            pltpu.VMEM((GCHUNKV,), jnp.int32),
            pltpu.VMEM((GCHUNKV,), jnp.float32),
            pltpu.SemaphoreType.DMA,
        ])
    def gather_kernel(tab_hbm, i_hbm, o_hbm, idx_v, rows_v, sem):
        wid = jax.lax.axis_index("s") * info.num_cores + jax.lax.axis_index("c")
        base = wid * per_w

        @pl.loop(0, n_chunks)
        def _(i):
            off = base + i * GCHUNKV
            pltpu.sync_copy(i_hbm.at[pl.ds(off, GCHUNKV)], idx_v)
            pltpu.async_copy(tab_hbm.at[idx_v], rows_v, sem).wait()
            pltpu.sync_copy(rows_v, o_hbm.at[pl.ds(off, GCHUNKV)])

    return gather_kernel(table, idx)


def _reduce_kernel(w_ref, gadv_ref, gval_ref, iv_ref, out_ref):
    NPK = HEADS * KNN
    w_adv = w_ref[:, :NPK]
    w_val = w_ref[:, NPK:]
    g = gadv_ref[...].reshape(RTILE, NPK, 128)[:, :, :VPAD]
    adv = jnp.sum(g * w_adv[:, :, None], axis=1)
    gv = gval_ref[...].reshape(RTILE, NPK)
    val = jnp.sum(gv * w_val, axis=1, keepdims=True)
    advd = adv[:, :NUM_ACTIONS]
    mean = jnp.sum(advd, axis=1, keepdims=True) * (1.0 / NUM_ACTIONS)
    out_ref[...] = advd - mean + val


# speedup vs baseline: 4.3961x; 1.2046x over previous
import functools
import jax
import jax.numpy as jnp
from jax.experimental import pallas as pl
from jax.experimental.pallas import tpu as pltpu
from jax.experimental.pallas import tpu_sc as plsc

B = 16384
HIDDEN = 256
NUM_ACTIONS = 18
HEADS = 4
K_DIM = 256
HALF = K_DIM // 2
N_KEYS = 1024
KNN = 64
N_VALUES = N_KEYS * N_KEYS

TILE = 128

_PAIRS = [(a, b) for a in range(KNN) for b in range(KNN // (a + 1))]
N_CAND = len(_PAIRS)
N_CPAD = 384

import numpy as _np
_A_MAT = _np.zeros((KNN, N_CPAD), _np.float32)
_B_MAT = _np.zeros((KNN, N_CPAD), _np.float32)
_PAD_MASK = _np.full((1, N_CPAD), -1e30, _np.float32)
for _c, (_a, _b) in enumerate(_PAIRS):
    _A_MAT[_a, _c] = 1.0
    _B_MAT[_b, _c] = 1.0
    _PAD_MASK[0, _c] = 0.0

_NEG = float('-inf')


def _extract_topk(s, k, n_sentinel):
    R, N = s.shape
    iota = jax.lax.broadcasted_iota(jnp.int32, (R, N), 1)
    oiota = jax.lax.broadcasted_iota(jnp.int32, (R, k), 1)

    def body(j, carry):
        cur, outv, outp = carry
        m = jnp.max(cur, axis=1, keepdims=True)
        hit = cur == m
        p = jnp.min(jnp.where(hit, iota, n_sentinel), axis=1, keepdims=True)
        cur = jnp.where(iota == p, _NEG, cur)
        sel = oiota == j
        outv = jnp.where(sel, m, outv)
        outp = jnp.where(sel, p, outp)
        return cur, outv, outp

    outv = jnp.full((R, k), _NEG, jnp.float32)
    outp = jnp.zeros((R, k), jnp.int32)
    _, outv, outp = jax.lax.fori_loop(0, k, body, (s, outv, outp))
    return outv, outp


def _extract_topk_payload_f(s, payload, k, n_sentinel):
    R, N = s.shape
    iota = jax.lax.broadcasted_iota(jnp.int32, (R, N), 1)
    oiota = jax.lax.broadcasted_iota(jnp.int32, (R, k), 1)
    big = jnp.float32(3e9)

    def body(j, carry):
        cur, outv, outi = carry
        m = jnp.max(cur, axis=1, keepdims=True)
        hit = cur == m
        p = jnp.min(jnp.where(hit, iota, n_sentinel), axis=1, keepdims=True)
        at_p = iota == p
        pay = jnp.min(jnp.where(at_p, payload, big), axis=1, keepdims=True)
        cur = jnp.where(at_p, _NEG, cur)
        sel = oiota == j
        outv = jnp.where(sel, m, outv)
        outi = jnp.where(sel, pay, outi)
        return cur, outv, outi

    outv = jnp.full((R, k), _NEG, jnp.float32)
    outi = jnp.zeros((R, k), jnp.float32)
    _, outv, outi = jax.lax.fori_loop(0, k, body, (s, outv, outi))
    return outv, outi


def _select_kernel(x_ref, wq_ref, bq_ref, k_adv_ref, k_val_ref,
                   ab_ref, mask_ref, w_ref, idx_ref):
    x = x_ref[...]
    q = jnp.dot(x, wq_ref[...], preferred_element_type=jnp.float32)
    q = q + bq_ref[0, :]
    ab = ab_ref[...]
    pad_mask = mask_ref[0, :]
    hi = jax.lax.Precision.HIGHEST

    s1s, s2s = [], []
    for pkm in range(2):
        k_ref = k_adv_ref if pkm == 0 else k_val_ref
        for h in range(HEADS):
            base = pkm * HEADS * K_DIM + h * K_DIM
            q1 = q[:, base:base + HALF]
            q2 = q[:, base + HALF:base + K_DIM]
            s1s.append(jax.lax.dot_general(
                q1, k_ref[2 * h], (((1,), (1,)), ((), ())),
                preferred_element_type=jnp.float32))
            s2s.append(jax.lax.dot_general(
                q2, k_ref[2 * h + 1], (((1,), (1,)), ((), ())),
                preferred_element_type=jnp.float32))

    s_all = jnp.concatenate(s1s + s2s, axis=0)
    sc_all, pos_all = _extract_topk(s_all, KNN, N_KEYS)

    half = 8 * TILE
    sc12 = jnp.concatenate([sc_all[:half], sc_all[half:]], axis=1)
    ix12 = jnp.concatenate(
        [pos_all[:half].astype(jnp.float32) * float(N_KEYS),
         pos_all[half:].astype(jnp.float32)], axis=1)
    cand_sc = jnp.dot(sc12, ab, precision=hi,
                      preferred_element_type=jnp.float32) + pad_mask
    cand_if = jnp.dot(ix12, ab, precision=hi,
                      preferred_element_type=jnp.float32)

    bsc, fidx_f = _extract_topk_payload_f(cand_sc, cand_if, KNN, N_CPAD)
    fidx = (fidx_f + 0.5).astype(jnp.int32)
    e = jnp.exp(bsc - bsc[:, :1])
    w = e / jnp.sum(e, axis=1, keepdims=True)
    w_ref[...] = w.reshape(8, TILE, KNN)
    idx_ref[...] = fidx.reshape(8, TILE, KNN)


def kernel(x, Wq_adv, bq_adv, keys_adv, values_adv, Wq_val, bq_val,
           keys_val, values_val):
    wq = jnp.concatenate([Wq_adv, Wq_val], axis=1)
    bq = jnp.concatenate([bq_adv, bq_val])[None, :]
    k_adv = keys_adv.reshape(HEADS * 2, N_KEYS, HALF)
    k_val = keys_val.reshape(HEADS * 2, N_KEYS, HALF)
    ab = jnp.asarray(_np.concatenate([_A_MAT, _B_MAT], axis=0))
    pad_mask = jnp.asarray(_PAD_MASK)

    grid = (B // TILE,)
    w_st, idx_st = pl.pallas_call(
        _select_kernel,
        grid=grid,
        in_specs=[
            pl.BlockSpec((TILE, HIDDEN), lambda i: (i, 0)),
            pl.BlockSpec((HIDDEN, 2 * HEADS * K_DIM), lambda i: (0, 0)),
            pl.BlockSpec((1, 2 * HEADS * K_DIM), lambda i: (0, 0)),
            pl.BlockSpec((HEADS * 2, N_KEYS, HALF), lambda i: (0, 0, 0)),
            pl.BlockSpec((HEADS * 2, N_KEYS, HALF), lambda i: (0, 0, 0)),
            pl.BlockSpec((2 * KNN, N_CPAD), lambda i: (0, 0)),
            pl.BlockSpec((1, N_CPAD), lambda i: (0, 0)),
        ],
        out_specs=[
            pl.BlockSpec((8, TILE, KNN), lambda i: (0, i, 0)),
            pl.BlockSpec((8, TILE, KNN), lambda i: (0, i, 0)),
        ],
        out_shape=[
            jax.ShapeDtypeStruct((8, B, KNN), jnp.float32),
            jax.ShapeDtypeStruct((8, B, KNN), jnp.int32),
        ],
    )(x, wq, bq, k_adv, k_val, ab, pad_mask)

    w_all = w_st.transpose(1, 0, 2).reshape(B, 2 * HEADS * KNN)
    idx_all = idx_st.transpose(1, 0, 2).reshape(B, 2 * HEADS * KNN)

    NPK = HEADS * KNN
    idx_adv = idx_all[:, :NPK].reshape(B * NPK)
    idx_val = idx_all[:, NPK:].reshape(B * NPK)
    vals_adv_pad = jnp.pad(values_adv, ((0, 0), (0, 128 - NUM_ACTIONS)))
    g_adv = _sc_gather(vals_adv_pad, idx_adv)
    g_val = _sc_gather_1d(values_val.reshape(N_VALUES), idx_val)
    g_val = g_val.reshape(B * NPK, 1)
    iv_val = idx_all[:, NPK:]

    out = pl.pallas_call(
        _reduce_kernel,
        grid=(B // RTILE,),
        in_specs=[
            pl.BlockSpec((RTILE, 2 * NPK), lambda i: (i, 0)),
            pl.BlockSpec((RTILE * NPK, 128), lambda i: (i, 0)),
            pl.BlockSpec((RTILE * NPK, 1), lambda i: (i, 0)),
            pl.BlockSpec((RTILE, NPK), lambda i: (i, 0)),
        ],
        out_specs=pl.BlockSpec((RTILE, NUM_ACTIONS), lambda i: (i, 0)),
        out_shape=jax.ShapeDtypeStruct((B, NUM_ACTIONS), jnp.float32),
    )(w_all, g_adv, g_val, iv_val)
    return out


VPAD = 32
VQ = 64
RTILE = 64
GCHUNK = 512
GCHUNKV = 2048


def _sc_gather(table, idx):
    m = idx.shape[0]
    info = plsc.get_sparse_core_info()
    nw = info.num_cores * info.num_subcores
    per_w = m // nw
    n_chunks = per_w // GCHUNK
    mesh = plsc.VectorSubcoreMesh(core_axis_name="c", subcore_axis_name="s")

    @functools.partial(
        pl.kernel,
        out_type=jax.ShapeDtypeStruct((m, 128), table.dtype),
        mesh=mesh,
        scratch_types=[
            pltpu.VMEM((GCHUNK,), jnp.int32),
            pltpu.VMEM((GCHUNK, 128), jnp.float32),
            pltpu.SemaphoreType.DMA,
        ])
    def gather_kernel(tab_hbm, i_hbm, o_hbm, idx_v, rows_v, sem):
        wid = jax.lax.axis_index("s") * info.num_cores + jax.lax.axis_index("c")
        base = wid * per_w

        @pl.loop(0, n_chunks)
        def _(i):
            off = base + i * GCHUNK
            pltpu.sync_copy(i_hbm.at[pl.ds(off, GCHUNK)], idx_v)
            pltpu.async_copy(tab_hbm.at[idx_v], rows_v, sem).wait()
            pltpu.sync_copy(rows_v, o_hbm.at[pl.ds(off, GCHUNK)])

    return gather_kernel(table, idx)


def _sc_gather_spmem(table, idx):
    m = idx.shape[0]
    n16 = table.shape[0]
    info = plsc.get_sparse_core_info()
    nw = info.num_cores * info.num_subcores
    per_w = m // nw
    n_chunks = per_w // GCHUNK
    mesh = plsc.VectorSubcoreMesh(core_axis_name="c", subcore_axis_name="s")

    @functools.partial(
        pl.kernel,
        out_type=jax.ShapeDtypeStruct((m, VQ), jnp.float32),
        mesh=mesh,
        scratch_types=[
            pltpu.VMEM((GCHUNK,), jnp.int32),
            pltpu.VMEM((GCHUNK, VQ), jnp.float32),
            pltpu.VMEM_SHARED((n16, VQ), jnp.float32),
            pltpu.SemaphoreType.DMA,
        ])
    def gather_kernel(tab_hbm, i_hbm, o_hbm, idx_v, rows_v, tab_sh, sem):
        sid = jax.lax.axis_index("s")

        @pl.when(sid == 0)
        def _():
            pltpu.sync_copy(tab_hbm, tab_sh)

        plsc.subcore_barrier()
        wid = sid * info.num_cores + jax.lax.axis_index("c")
        base = wid * per_w

        @pl.loop(0, n_chunks)
        def _(i):
            off = base + i * GCHUNK
            pltpu.sync_copy(i_hbm.at[pl.ds(off, GCHUNK)], idx_v)
            pltpu.async_copy(tab_sh.at[idx_v], rows_v, sem).wait()
            pltpu.sync_copy(rows_v, o_hbm.at[pl.ds(off, GCHUNK)])

    return gather_kernel(table, idx)


def _sc_gather_1d(table, idx):
    m = idx.shape[0]
    info = plsc.get_sparse_core_info()
    nw = info.num_cores * info.num_subcores
    per_w = m // nw
    n_chunks = per_w // GCHUNKV
    mesh = plsc.VectorSubcoreMesh(core_axis_name="c", subcore_axis_name="s")

    @functools.partial(
        pl.kernel,
        out_type=jax.ShapeDtypeStruct((m,), jnp.float32),
        mesh=mesh,
        scratch_types=[
            pltpu.VMEM((GCHUNKV,), jnp.int32),
            pltpu.VMEM((GCHUNKV,), jnp.float32),
            pltpu.SemaphoreType.DMA,
        ])
    def gather_kernel(tab_hbm, i_hbm, o_hbm, idx_v, rows_v, sem):
        wid = jax.lax.axis_index("s") * info.num_cores + jax.lax.axis_index("c")
        base = wid * per_w

        @pl.loop(0, n_chunks)
        def _(i):
            off = base + i * GCHUNKV
            pltpu.sync_copy(i_hbm.at[pl.ds(off, GCHUNKV)], idx_v)
            pltpu.async_copy(tab_hbm.at[idx_v], rows_v, sem).wait()
            pltpu.sync_copy(rows_v, o_hbm.at[pl.ds(off, GCHUNKV)])

    return gather_kernel(table, idx)


def _reduce_kernel(w_ref, gadv_ref, gval_ref, iv_ref, out_ref):
    NPK = HEADS * KNN
    w_adv = w_ref[:, :NPK]
    w_val = w_ref[:, NPK:]
    g = gadv_ref[...].reshape(RTILE, NPK, 128)[:, :, :VPAD]
    adv = jnp.sum(g * w_adv[:, :, None], axis=1)
    gv = gval_ref[...].reshape(RTILE, NPK)
    val = jnp.sum(gv * w_val, axis=1, keepdims=True)
    advd = adv[:, :NUM_ACTIONS]
    mean = jnp.sum(advd, axis=1, keepdims=True) * (1.0 / NUM_ACTIONS)
    out_ref[...] = advd - mean + val


# speedup vs baseline: 8.2694x; 1.8811x over previous
import functools
import jax
import jax.numpy as jnp
from jax.experimental import pallas as pl
from jax.experimental.pallas import tpu as pltpu
from jax.experimental.pallas import tpu_sc as plsc

B = 16384
HIDDEN = 256
NUM_ACTIONS = 18
HEADS = 4
K_DIM = 256
HALF = K_DIM // 2
N_KEYS = 1024
KNN = 64
N_VALUES = N_KEYS * N_KEYS

TILE = 128

_PAIRS = [(a, b) for a in range(KNN) for b in range(KNN // (a + 1))]
N_CAND = len(_PAIRS)
N_CPAD = 384

import numpy as _np
_A_MAT = _np.zeros((KNN, N_CPAD), _np.float32)
_B_MAT = _np.zeros((KNN, N_CPAD), _np.float32)
_PAD_MASK = _np.full((1, N_CPAD), -1e30, _np.float32)
for _c, (_a, _b) in enumerate(_PAIRS):
    _A_MAT[_a, _c] = 1.0
    _B_MAT[_b, _c] = 1.0
    _PAD_MASK[0, _c] = 0.0

_NEG = float('-inf')


def _extract_topk(s, k, n_sentinel):
    R, N = s.shape
    iota = jax.lax.broadcasted_iota(jnp.int32, (R, N), 1)
    oiota = jax.lax.broadcasted_iota(jnp.int32, (R, k), 1)

    def body(j, carry):
        cur, outv, outp = carry
        m = jnp.max(cur, axis=1, keepdims=True)
        hit = cur == m
        p = jnp.min(jnp.where(hit, iota, n_sentinel), axis=1, keepdims=True)
        cur = jnp.where(iota == p, _NEG, cur)
        sel = oiota == j
        outv = jnp.where(sel, m, outv)
        outp = jnp.where(sel, p, outp)
        return cur, outv, outp

    outv = jnp.full((R, k), _NEG, jnp.float32)
    outp = jnp.zeros((R, k), jnp.int32)
    _, outv, outp = jax.lax.fori_loop(0, k, body, (s, outv, outp))
    return outv, outp


def _extract_topk_payload_f(s, payload, k, n_sentinel):
    R, N = s.shape
    iota = jax.lax.broadcasted_iota(jnp.int32, (R, N), 1)
    oiota = jax.lax.broadcasted_iota(jnp.int32, (R, k), 1)
    big = jnp.float32(3e9)

    def body(j, carry):
        cur, outv, outi = carry
        m = jnp.max(cur, axis=1, keepdims=True)
        hit = cur == m
        p = jnp.min(jnp.where(hit, iota, n_sentinel), axis=1, keepdims=True)
        at_p = iota == p
        pay = jnp.min(jnp.where(at_p, payload, big), axis=1, keepdims=True)
        cur = jnp.where(at_p, _NEG, cur)
        sel = oiota == j
        outv = jnp.where(sel, m, outv)
        outi = jnp.where(sel, pay, outi)
        return cur, outv, outi

    outv = jnp.full((R, k), _NEG, jnp.float32)
    outi = jnp.zeros((R, k), jnp.float32)
    _, outv, outi = jax.lax.fori_loop(0, k, body, (s, outv, outi))
    return outv, outi


def _select_kernel(x_ref, wq_ref, bq_ref, k_adv_ref, k_val_ref,
                   ab_ref, mask_ref, w_ref, idx_ref):
    x = x_ref[...]
    q = jnp.dot(x, wq_ref[...], preferred_element_type=jnp.float32)
    q = q + bq_ref[0, :]
    ab = ab_ref[...]
    pad_mask = mask_ref[0, :]
    hi = jax.lax.Precision.HIGHEST

    s1s, s2s = [], []
    for pkm in range(2):
        k_ref = k_adv_ref if pkm == 0 else k_val_ref
        for h in range(HEADS):
            base = pkm * HEADS * K_DIM + h * K_DIM
            q1 = q[:, base:base + HALF]
            q2 = q[:, base + HALF:base + K_DIM]
            s1s.append(jax.lax.dot_general(
                q1, k_ref[2 * h], (((1,), (1,)), ((), ())),
                preferred_element_type=jnp.float32))
            s2s.append(jax.lax.dot_general(
                q2, k_ref[2 * h + 1], (((1,), (1,)), ((), ())),
                preferred_element_type=jnp.float32))

    s_all = jnp.concatenate(s1s + s2s, axis=0)
    sc_all, pos_all = _extract_topk(s_all, KNN, N_KEYS)

    half = 8 * TILE
    sc12 = jnp.concatenate([sc_all[:half], sc_all[half:]], axis=1)
    ix12 = jnp.concatenate(
        [pos_all[:half].astype(jnp.float32) * float(N_KEYS),
         pos_all[half:].astype(jnp.float32)], axis=1)
    cand_sc = jnp.dot(sc12, ab, precision=hi,
                      preferred_element_type=jnp.float32) + pad_mask
    cand_if = jnp.dot(ix12, ab, precision=hi,
                      preferred_element_type=jnp.float32)

    bsc, fidx_f = _extract_topk_payload_f(cand_sc, cand_if, KNN, N_CPAD)
    fidx = (fidx_f + 0.5).astype(jnp.int32)
    e = jnp.exp(bsc - bsc[:, :1])
    w = e / jnp.sum(e, axis=1, keepdims=True)
    w_ref[...] = w.reshape(8, TILE, KNN)
    idx_ref[...] = fidx.reshape(8, TILE, KNN)


def kernel(x, Wq_adv, bq_adv, keys_adv, values_adv, Wq_val, bq_val,
           keys_val, values_val):
    devs = jax.devices()
    if len(devs) >= 2:
        mesh = jax.sharding.Mesh(_np.array(devs[:2]), ("d",))
        P = jax.sharding.PartitionSpec
        rep = P()
        f = jax.shard_map(
            _kernel_impl, mesh=mesh,
            in_specs=(P("d"), rep, rep, rep, rep, rep, rep, rep, rep),
            out_specs=P("d"), check_vma=False)
        return f(x, Wq_adv, bq_adv, keys_adv, values_adv, Wq_val, bq_val,
                 keys_val, values_val)
    return _kernel_impl(x, Wq_adv, bq_adv, keys_adv, values_adv, Wq_val,
                        bq_val, keys_val, values_val)


def _kernel_impl(x, Wq_adv, bq_adv, keys_adv, values_adv, Wq_val, bq_val,
                 keys_val, values_val):
    bs = x.shape[0]
    wq = jnp.concatenate([Wq_adv, Wq_val], axis=1)
    bq = jnp.concatenate([bq_adv, bq_val])[None, :]
    k_adv = keys_adv.reshape(HEADS * 2, N_KEYS, HALF)
    k_val = keys_val.reshape(HEADS * 2, N_KEYS, HALF)
    ab = jnp.asarray(_np.concatenate([_A_MAT, _B_MAT], axis=0))
    pad_mask = jnp.asarray(_PAD_MASK)

    grid = (bs // TILE,)
    w_st, idx_st = pl.pallas_call(
        _select_kernel,
        grid=grid,
        in_specs=[
            pl.BlockSpec((TILE, HIDDEN), lambda i: (i, 0)),
            pl.BlockSpec((HIDDEN, 2 * HEADS * K_DIM), lambda i: (0, 0)),
            pl.BlockSpec((1, 2 * HEADS * K_DIM), lambda i: (0, 0)),
            pl.BlockSpec((HEADS * 2, N_KEYS, HALF), lambda i: (0, 0, 0)),
            pl.BlockSpec((HEADS * 2, N_KEYS, HALF), lambda i: (0, 0, 0)),
            pl.BlockSpec((2 * KNN, N_CPAD), lambda i: (0, 0)),
            pl.BlockSpec((1, N_CPAD), lambda i: (0, 0)),
        ],
        out_specs=[
            pl.BlockSpec((8, TILE, KNN), lambda i: (0, i, 0)),
            pl.BlockSpec((8, TILE, KNN), lambda i: (0, i, 0)),
        ],
        out_shape=[
            jax.ShapeDtypeStruct((8, bs, KNN), jnp.float32),
            jax.ShapeDtypeStruct((8, bs, KNN), jnp.int32),
        ],
    )(x, wq, bq, k_adv, k_val, ab, pad_mask)

    w_all = w_st.transpose(1, 0, 2).reshape(bs, 2 * HEADS * KNN)
    idx_all = idx_st.transpose(1, 0, 2).reshape(bs, 2 * HEADS * KNN)

    NPK = HEADS * KNN
    idx_adv = idx_all[:, :NPK].reshape(bs * NPK)
    idx_val = idx_all[:, NPK:].reshape(bs * NPK)
    vals_adv_pad = jnp.pad(values_adv, ((0, 0), (0, 128 - NUM_ACTIONS)))
    g_adv = _sc_gather(vals_adv_pad, idx_adv)
    g_val = _sc_gather_1d(values_val.reshape(N_VALUES), idx_val)
    g_val = g_val.reshape(bs * NPK, 1)
    iv_val = idx_all[:, NPK:]

    out = pl.pallas_call(
        _reduce_kernel,
        grid=(bs // RTILE,),
        in_specs=[
            pl.BlockSpec((RTILE, 2 * NPK), lambda i: (i, 0)),
            pl.BlockSpec((RTILE * NPK, 128), lambda i: (i, 0)),
            pl.BlockSpec((RTILE * NPK, 1), lambda i: (i, 0)),
            pl.BlockSpec((RTILE, NPK), lambda i: (i, 0)),
        ],
        out_specs=pl.BlockSpec((RTILE, NUM_ACTIONS), lambda i: (i, 0)),
        out_shape=jax.ShapeDtypeStruct((bs, NUM_ACTIONS), jnp.float32),
    )(w_all, g_adv, g_val, iv_val)
    return out


VPAD = 32
VQ = 64
RTILE = 64
GCHUNK = 512
GCHUNKV = 2048


def _sc_gather(table, idx):
    m = idx.shape[0]
    info = plsc.get_sparse_core_info()
    nw = info.num_cores * info.num_subcores
    per_w = m // nw
    n_chunks = per_w // GCHUNK
    mesh = plsc.VectorSubcoreMesh(core_axis_name="c", subcore_axis_name="s")

    @functools.partial(
        pl.kernel,
        out_type=jax.ShapeDtypeStruct((m, 128), table.dtype),
        mesh=mesh,
        scratch_types=[
            pltpu.VMEM((GCHUNK,), jnp.int32),
            pltpu.VMEM((GCHUNK, 128), jnp.float32),
            pltpu.SemaphoreType.DMA,
        ])
    def gather_kernel(tab_hbm, i_hbm, o_hbm, idx_v, rows_v, sem):
        wid = jax.lax.axis_index("s") * info.num_cores + jax.lax.axis_index("c")
        base = wid * per_w

        @pl.loop(0, n_chunks)
        def _(i):
            off = base + i * GCHUNK
            pltpu.sync_copy(i_hbm.at[pl.ds(off, GCHUNK)], idx_v)
            pltpu.async_copy(tab_hbm.at[idx_v], rows_v, sem).wait()
            pltpu.sync_copy(rows_v, o_hbm.at[pl.ds(off, GCHUNK)])

    return gather_kernel(table, idx)


def _sc_gather_spmem(table, idx):
    m = idx.shape[0]
    n16 = table.shape[0]
    info = plsc.get_sparse_core_info()
    nw = info.num_cores * info.num_subcores
    per_w = m // nw
    n_chunks = per_w // GCHUNK
    mesh = plsc.VectorSubcoreMesh(core_axis_name="c", subcore_axis_name="s")

    @functools.partial(
        pl.kernel,
        out_type=jax.ShapeDtypeStruct((m, VQ), jnp.float32),
        mesh=mesh,
        scratch_types=[
            pltpu.VMEM((GCHUNK,), jnp.int32),
            pltpu.VMEM((GCHUNK, VQ), jnp.float32),
            pltpu.VMEM_SHARED((n16, VQ), jnp.float32),
            pltpu.SemaphoreType.DMA,
        ])
    def gather_kernel(tab_hbm, i_hbm, o_hbm, idx_v, rows_v, tab_sh, sem):
        sid = jax.lax.axis_index("s")

        @pl.when(sid == 0)
        def _():
            pltpu.sync_copy(tab_hbm, tab_sh)

        plsc.subcore_barrier()
        wid = sid * info.num_cores + jax.lax.axis_index("c")
        base = wid * per_w

        @pl.loop(0, n_chunks)
        def _(i):
            off = base + i * GCHUNK
            pltpu.sync_copy(i_hbm.at[pl.ds(off, GCHUNK)], idx_v)
            pltpu.async_copy(tab_sh.at[idx_v], rows_v, sem).wait()
            pltpu.sync_copy(rows_v, o_hbm.at[pl.ds(off, GCHUNK)])

    return gather_kernel(table, idx)


def _sc_gather_1d(table, idx):
    m = idx.shape[0]
    info = plsc.get_sparse_core_info()
    nw = info.num_cores * info.num_subcores
    per_w = m // nw
    n_chunks = per_w // GCHUNKV
    mesh = plsc.VectorSubcoreMesh(core_axis_name="c", subcore_axis_name="s")

    @functools.partial(
        pl.kernel,
        out_type=jax.ShapeDtypeStruct((m,), jnp.float32),
        mesh=mesh,
        scratch_types=[
            pltpu.VMEM((GCHUNKV,), jnp.int32),
            pltpu.VMEM((GCHUNKV,), jnp.float32),
            pltpu.SemaphoreType.DMA,
        ])
    def gather_kernel(tab_hbm, i_hbm, o_hbm, idx_v, rows_v, sem):
        wid = jax.lax.axis_index("s") * info.num_cores + jax.lax.axis_index("c")
        base = wid * per_w

        @pl.loop(0, n_chunks)
        def _(i):
            off = base + i * GCHUNKV
            pltpu.sync_copy(i_hbm.at[pl.ds(off, GCHUNKV)], idx_v)
            pltpu.async_copy(tab_hbm.at[idx_v], rows_v, sem).wait()
            pltpu.sync_copy(rows_v, o_hbm.at[pl.ds(off, GCHUNKV)])

    return gather_kernel(table, idx)


def _reduce_kernel(w_ref, gadv_ref, gval_ref, iv_ref, out_ref):
    NPK = HEADS * KNN
    w_adv = w_ref[:, :NPK]
    w_val = w_ref[:, NPK:]
    g = gadv_ref[...].reshape(RTILE, NPK, 128)[:, :, :VPAD]
    adv = jnp.sum(g * w_adv[:, :, None], axis=1)
    gv = gval_ref[...].reshape(RTILE, NPK)
    val = jnp.sum(gv * w_val, axis=1, keepdims=True)
    advd = adv[:, :NUM_ACTIONS]
    mean = jnp.sum(advd, axis=1, keepdims=True) * (1.0 / NUM_ACTIONS)
    out_ref[...] = advd - mean + val


# speedup vs baseline: 10.0814x; 1.2191x over previous
import functools
import jax
import jax.numpy as jnp
from jax.experimental import pallas as pl
from jax.experimental.pallas import tpu as pltpu
from jax.experimental.pallas import tpu_sc as plsc

B = 16384
HIDDEN = 256
NUM_ACTIONS = 18
HEADS = 4
K_DIM = 256
HALF = K_DIM // 2
N_KEYS = 1024
KNN = 64
N_VALUES = N_KEYS * N_KEYS

TILE = 128

_PAIRS = [(a, b) for a in range(KNN) for b in range(KNN // (a + 1))]
N_CAND = len(_PAIRS)
N_CPAD = 384

import numpy as _np
_A_MAT = _np.zeros((KNN, N_CPAD), _np.float32)
_B_MAT = _np.zeros((KNN, N_CPAD), _np.float32)
_PAD_MASK = _np.full((1, N_CPAD), -1e30, _np.float32)
for _c, (_a, _b) in enumerate(_PAIRS):
    _A_MAT[_a, _c] = 1.0
    _B_MAT[_b, _c] = 1.0
    _PAD_MASK[0, _c] = 0.0

_NEG = float('-inf')


def _extract_topk(s, k, n_sentinel):
    R, N = s.shape
    iota = jax.lax.broadcasted_iota(jnp.int32, (R, N), 1)
    oiota = jax.lax.broadcasted_iota(jnp.int32, (R, k), 1)

    def body(j, carry):
        cur, outv, outp = carry
        m = jnp.max(cur, axis=1, keepdims=True)
        hit = cur == m
        p = jnp.min(jnp.where(hit, iota, n_sentinel), axis=1, keepdims=True)
        cur = jnp.where(iota == p, _NEG, cur)
        sel = oiota == j
        outv = jnp.where(sel, m, outv)
        outp = jnp.where(sel, p, outp)
        return cur, outv, outp

    outv = jnp.full((R, k), _NEG, jnp.float32)
    outp = jnp.zeros((R, k), jnp.int32)
    _, outv, outp = jax.lax.fori_loop(0, k, body, (s, outv, outp),
                                      unroll=2)
    return outv, outp


def _extract_topk_payload_f(s, payload, k, n_sentinel):
    R, N = s.shape
    iota = jax.lax.broadcasted_iota(jnp.int32, (R, N), 1)
    oiota = jax.lax.broadcasted_iota(jnp.int32, (R, k), 1)
    big = jnp.float32(3e9)

    def body(j, carry):
        cur, outv, outi = carry
        m = jnp.max(cur, axis=1, keepdims=True)
        hit = cur == m
        p = jnp.min(jnp.where(hit, iota, n_sentinel), axis=1, keepdims=True)
        at_p = iota == p
        pay = jnp.min(jnp.where(at_p, payload, big), axis=1, keepdims=True)
        cur = jnp.where(at_p, _NEG, cur)
        sel = oiota == j
        outv = jnp.where(sel, m, outv)
        outi = jnp.where(sel, pay, outi)
        return cur, outv, outi

    outv = jnp.full((R, k), _NEG, jnp.float32)
    outi = jnp.zeros((R, k), jnp.float32)
    _, outv, outi = jax.lax.fori_loop(0, k, body, (s, outv, outi),
                                      unroll=2)
    return outv, outi


def _select_kernel(x_ref, wq_ref, bq_ref, k_adv_ref, k_val_ref,
                   ab_ref, mask_ref, w_ref, idx_ref):
    x = x_ref[...]
    q = jnp.dot(x, wq_ref[...], preferred_element_type=jnp.float32)
    q = q + bq_ref[0, :]
    ab = ab_ref[...]
    pad_mask = mask_ref[0, :]
    hi = jax.lax.Precision.HIGHEST

    s1s, s2s = [], []
    for pkm in range(2):
        k_ref = k_adv_ref if pkm == 0 else k_val_ref
        for h in range(HEADS):
            base = pkm * HEADS * K_DIM + h * K_DIM
            q1 = q[:, base:base + HALF]
            q2 = q[:, base + HALF:base + K_DIM]
            s1s.append(jax.lax.dot_general(
                q1, k_ref[2 * h], (((1,), (1,)), ((), ())),
                preferred_element_type=jnp.float32))
            s2s.append(jax.lax.dot_general(
                q2, k_ref[2 * h + 1], (((1,), (1,)), ((), ())),
                preferred_element_type=jnp.float32))

    s_all = jnp.concatenate(s1s + s2s, axis=0)
    sc_all, pos_all = _extract_topk(s_all, KNN, N_KEYS)

    half = 8 * TILE
    sc12 = jnp.concatenate([sc_all[:half], sc_all[half:]], axis=1)
    ix12 = jnp.concatenate(
        [pos_all[:half].astype(jnp.float32) * float(N_KEYS),
         pos_all[half:].astype(jnp.float32)], axis=1)
    cand_sc = jnp.dot(sc12, ab, precision=hi,
                      preferred_element_type=jnp.float32) + pad_mask
    cand_if = jnp.dot(ix12, ab, precision=hi,
                      preferred_element_type=jnp.float32)

    bsc, fidx_f = _extract_topk_payload_f(cand_sc, cand_if, KNN, N_CPAD)
    fidx = (fidx_f + 0.5).astype(jnp.int32)
    e = jnp.exp(bsc - bsc[:, :1])
    w = e / jnp.sum(e, axis=1, keepdims=True)
    w_ref[...] = w.reshape(8, TILE, KNN)
    idx_ref[...] = fidx.reshape(8, TILE, KNN)


def kernel(x, Wq_adv, bq_adv, keys_adv, values_adv, Wq_val, bq_val,
           keys_val, values_val):
    devs = jax.devices()
    if len(devs) >= 2:
        mesh = jax.sharding.Mesh(_np.array(devs[:2]), ("d",))
        P = jax.sharding.PartitionSpec
        rep = P()
        f = jax.shard_map(
            _kernel_impl, mesh=mesh,
            in_specs=(P("d"), rep, rep, rep, rep, rep, rep, rep, rep),
            out_specs=P("d"), check_vma=False)
        return f(x, Wq_adv, bq_adv, keys_adv, values_adv, Wq_val, bq_val,
                 keys_val, values_val)
    return _kernel_impl(x, Wq_adv, bq_adv, keys_adv, values_adv, Wq_val,
                        bq_val, keys_val, values_val)


def _kernel_impl(x, Wq_adv, bq_adv, keys_adv, values_adv, Wq_val, bq_val,
                 keys_val, values_val):
    bs = x.shape[0]
    wq = jnp.concatenate([Wq_adv, Wq_val], axis=1)
    bq = jnp.concatenate([bq_adv, bq_val])[None, :]
    k_adv = keys_adv.reshape(HEADS * 2, N_KEYS, HALF)
    k_val = keys_val.reshape(HEADS * 2, N_KEYS, HALF)
    ab = jnp.asarray(_np.concatenate([_A_MAT, _B_MAT], axis=0))
    pad_mask = jnp.asarray(_PAD_MASK)

    grid = (bs // TILE,)
    w_st, idx_st = pl.pallas_call(
        _select_kernel,
        grid=grid,
        in_specs=[
            pl.BlockSpec((TILE, HIDDEN), lambda i: (i, 0)),
            pl.BlockSpec((HIDDEN, 2 * HEADS * K_DIM), lambda i: (0, 0)),
            pl.BlockSpec((1, 2 * HEADS * K_DIM), lambda i: (0, 0)),
            pl.BlockSpec((HEADS * 2, N_KEYS, HALF), lambda i: (0, 0, 0)),
            pl.BlockSpec((HEADS * 2, N_KEYS, HALF), lambda i: (0, 0, 0)),
            pl.BlockSpec((2 * KNN, N_CPAD), lambda i: (0, 0)),
            pl.BlockSpec((1, N_CPAD), lambda i: (0, 0)),
        ],
        out_specs=[
            pl.BlockSpec((8, TILE, KNN), lambda i: (0, i, 0)),
            pl.BlockSpec((8, TILE, KNN), lambda i: (0, i, 0)),
        ],
        out_shape=[
            jax.ShapeDtypeStruct((8, bs, KNN), jnp.float32),
            jax.ShapeDtypeStruct((8, bs, KNN), jnp.int32),
        ],
    )(x, wq, bq, k_adv, k_val, ab, pad_mask)

    w_all = w_st.transpose(1, 0, 2).reshape(bs, 2 * HEADS * KNN)
    idx_all = idx_st.transpose(1, 0, 2).reshape(bs, 2 * HEADS * KNN)

    NPK = HEADS * KNN
    idx_adv = idx_all[:, :NPK].reshape(bs * NPK)
    idx_val = idx_all[:, NPK:].reshape(bs * NPK)
    vals_adv_pad = jnp.pad(values_adv, ((0, 0), (0, 128 - NUM_ACTIONS)))
    g_adv = _sc_gather(vals_adv_pad, idx_adv)
    g_val = _sc_gather_1d(values_val.reshape(N_VALUES), idx_val)
    g_val = g_val.reshape(bs * NPK, 1)
    iv_val = idx_all[:, NPK:]

    out = pl.pallas_call(
        _reduce_kernel,
        grid=(bs // RTILE,),
        in_specs=[
            pl.BlockSpec((RTILE, 2 * NPK), lambda i: (i, 0)),
            pl.BlockSpec((RTILE * NPK, 128), lambda i: (i, 0)),
            pl.BlockSpec((RTILE * NPK, 1), lambda i: (i, 0)),
            pl.BlockSpec((RTILE, NPK), lambda i: (i, 0)),
        ],
        out_specs=pl.BlockSpec((RTILE, NUM_ACTIONS), lambda i: (i, 0)),
        out_shape=jax.ShapeDtypeStruct((bs, NUM_ACTIONS), jnp.float32),
    )(w_all, g_adv, g_val, iv_val)
    return out


VPAD = 32
VQ = 64
RTILE = 64
GCHUNK = 512
GCHUNKV = 2048


def _sc_gather(table, idx):
    m = idx.shape[0]
    info = plsc.get_sparse_core_info()
    nw = info.num_cores * info.num_subcores
    per_w = m // nw
    n_chunks = per_w // GCHUNK
    mesh = plsc.VectorSubcoreMesh(core_axis_name="c", subcore_axis_name="s")

    @functools.partial(
        pl.kernel,
        out_type=jax.ShapeDtypeStruct((m, 128), table.dtype),
        mesh=mesh,
        scratch_types=[
            pltpu.VMEM((GCHUNK,), jnp.int32),
            pltpu.VMEM((GCHUNK, 128), jnp.float32),
            pltpu.SemaphoreType.DMA,
        ])
    def gather_kernel(tab_hbm, i_hbm, o_hbm, idx_v, rows_v, sem):
        wid = jax.lax.axis_index("s") * info.num_cores + jax.lax.axis_index("c")
        base = wid * per_w

        @pl.loop(0, n_chunks)
        def _(i):
            off = base + i * GCHUNK
            pltpu.sync_copy(i_hbm.at[pl.ds(off, GCHUNK)], idx_v)
            pltpu.async_copy(tab_hbm.at[idx_v], rows_v, sem).wait()
            pltpu.sync_copy(rows_v, o_hbm.at[pl.ds(off, GCHUNK)])

    return gather_kernel(table, idx)


def _sc_gather_spmem(table, idx):
    m = idx.shape[0]
    n16 = table.shape[0]
    info = plsc.get_sparse_core_info()
    nw = info.num_cores * info.num_subcores
    per_w = m // nw
    n_chunks = per_w // GCHUNK
    mesh = plsc.VectorSubcoreMesh(core_axis_name="c", subcore_axis_name="s")

    @functools.partial(
        pl.kernel,
        out_type=jax.ShapeDtypeStruct((m, VQ), jnp.float32),
        mesh=mesh,
        scratch_types=[
            pltpu.VMEM((GCHUNK,), jnp.int32),
            pltpu.VMEM((GCHUNK, VQ), jnp.float32),
            pltpu.VMEM_SHARED((n16, VQ), jnp.float32),
            pltpu.SemaphoreType.DMA,
        ])
    def gather_kernel(tab_hbm, i_hbm, o_hbm, idx_v, rows_v, tab_sh, sem):
        sid = jax.lax.axis_index("s")

        @pl.when(sid == 0)
        def _():
            pltpu.sync_copy(tab_hbm, tab_sh)

        plsc.subcore_barrier()
        wid = sid * info.num_cores + jax.lax.axis_index("c")
        base = wid * per_w

        @pl.loop(0, n_chunks)
        def _(i):
            off = base + i * GCHUNK
            pltpu.sync_copy(i_hbm.at[pl.ds(off, GCHUNK)], idx_v)
            pltpu.async_copy(tab_sh.at[idx_v], rows_v, sem).wait()
            pltpu.sync_copy(rows_v, o_hbm.at[pl.ds(off, GCHUNK)])

    return gather_kernel(table, idx)


def _sc_gather_1d(table, idx):
    m = idx.shape[0]
    info = plsc.get_sparse_core_info()
    nw = info.num_cores * info.num_subcores
    per_w = m // nw
    n_chunks = per_w // GCHUNKV
    mesh = plsc.VectorSubcoreMesh(core_axis_name="c", subcore_axis_name="s")

    @functools.partial(
        pl.kernel,
        out_type=jax.ShapeDtypeStruct((m,), jnp.float32),
        mesh=mesh,
        scratch_types=[
            pltpu.VMEM((GCHUNKV,), jnp.int32),
            pltpu.VMEM((GCHUNKV,), jnp.float32),
            pltpu.SemaphoreType.DMA,
        ])
    def gather_kernel(tab_hbm, i_hbm, o_hbm, idx_v, rows_v, sem):
        wid = jax.lax.axis_index("s") * info.num_cores + jax.lax.axis_index("c")
        base = wid * per_w

        @pl.loop(0, n_chunks)
        def _(i):
            off = base + i * GCHUNKV
            pltpu.sync_copy(i_hbm.at[pl.ds(off, GCHUNKV)], idx_v)
            pltpu.async_copy(tab_hbm.at[idx_v], rows_v, sem).wait()
            pltpu.sync_copy(rows_v, o_hbm.at[pl.ds(off, GCHUNKV)])

    return gather_kernel(table, idx)


def _reduce_kernel(w_ref, gadv_ref, gval_ref, iv_ref, out_ref):
    NPK = HEADS * KNN
    w_adv = w_ref[:, :NPK]
    w_val = w_ref[:, NPK:]
    g = gadv_ref[...].reshape(RTILE, NPK, 128)[:, :, :VPAD]
    adv = jnp.sum(g * w_adv[:, :, None], axis=1)
    gv = gval_ref[...].reshape(RTILE, NPK)
    val = jnp.sum(gv * w_val, axis=1, keepdims=True)
    advd = adv[:, :NUM_ACTIONS]
    mean = jnp.sum(advd, axis=1, keepdims=True) * (1.0 / NUM_ACTIONS)
    out_ref[...] = advd - mean + val


# speedup vs baseline: 11.3178x; 1.1226x over previous
import functools
import jax
import jax.numpy as jnp
from jax.experimental import pallas as pl
from jax.experimental.pallas import tpu as pltpu
from jax.experimental.pallas import tpu_sc as plsc

B = 16384
HIDDEN = 256
NUM_ACTIONS = 18
HEADS = 4
K_DIM = 256
HALF = K_DIM // 2
N_KEYS = 1024
KNN = 64
N_VALUES = N_KEYS * N_KEYS

TILE = 128

_PAIRS = [(a, b) for a in range(KNN) for b in range(KNN // (a + 1))]
N_CAND = len(_PAIRS)
N_CPAD = 384

import numpy as _np
_A_MAT = _np.zeros((KNN, N_CPAD), _np.float32)
_B_MAT = _np.zeros((KNN, N_CPAD), _np.float32)
_PAD_MASK = _np.full((1, N_CPAD), -1e30, _np.float32)
for _c, (_a, _b) in enumerate(_PAIRS):
    _A_MAT[_a, _c] = 1.0
    _B_MAT[_b, _c] = 1.0
    _PAD_MASK[0, _c] = 0.0

_NEG = float('-inf')


def _extract_topk(s, k, n_sentinel):
    R, N = s.shape
    iota = jax.lax.broadcasted_iota(jnp.int32, (R, N), 1)
    oiota = jax.lax.broadcasted_iota(jnp.int32, (R, k), 1)

    def body(j, carry):
        cur, outv, outp = carry
        m = jnp.max(cur, axis=1, keepdims=True)
        hit = cur == m
        p = jnp.min(jnp.where(hit, iota, n_sentinel), axis=1, keepdims=True)
        cur = jnp.where(iota == p, _NEG, cur)
        sel = oiota == j
        outv = jnp.where(sel, m, outv)
        outp = jnp.where(sel, p, outp)
        return cur, outv, outp

    outv = jnp.full((R, k), _NEG, jnp.float32)
    outp = jnp.zeros((R, k), jnp.int32)
    _, outv, outp = jax.lax.fori_loop(0, k, body, (s, outv, outp),
                                      unroll=4)
    return outv, outp


def _extract_topk_payload_f(s, payload, k, n_sentinel):
    R, N = s.shape
    iota = jax.lax.broadcasted_iota(jnp.int32, (R, N), 1)
    oiota = jax.lax.broadcasted_iota(jnp.int32, (R, k), 1)
    big = jnp.float32(3e9)

    def body(j, carry):
        cur, outv, outi = carry
        m = jnp.max(cur, axis=1, keepdims=True)
        hit = cur == m
        p = jnp.min(jnp.where(hit, iota, n_sentinel), axis=1, keepdims=True)
        at_p = iota == p
        pay = jnp.min(jnp.where(at_p, payload, big), axis=1, keepdims=True)
        cur = jnp.where(at_p, _NEG, cur)
        sel = oiota == j
        outv = jnp.where(sel, m, outv)
        outi = jnp.where(sel, pay, outi)
        return cur, outv, outi

    outv = jnp.full((R, k), _NEG, jnp.float32)
    outi = jnp.zeros((R, k), jnp.float32)
    _, outv, outi = jax.lax.fori_loop(0, k, body, (s, outv, outi),
                                      unroll=4)
    return outv, outi


def _select_kernel(x_ref, wq_ref, bq_ref, k_adv_ref, k_val_ref,
                   ab_ref, mask_ref, w_ref, idx_ref):
    x = x_ref[...]
    q = jnp.dot(x, wq_ref[...], preferred_element_type=jnp.float32)
    q = q + bq_ref[0, :]
    ab = ab_ref[...]
    pad_mask = mask_ref[0, :]
    hi = jax.lax.Precision.HIGHEST

    s1s, s2s = [], []
    for pkm in range(2):
        k_ref = k_adv_ref if pkm == 0 else k_val_ref
        for h in range(HEADS):
            base = pkm * HEADS * K_DIM + h * K_DIM
            q1 = q[:, base:base + HALF]
            q2 = q[:, base + HALF:base + K_DIM]
            s1s.append(jax.lax.dot_general(
                q1, k_ref[2 * h], (((1,), (1,)), ((), ())),
                preferred_element_type=jnp.float32))
            s2s.append(jax.lax.dot_general(
                q2, k_ref[2 * h + 1], (((1,), (1,)), ((), ())),
                preferred_element_type=jnp.float32))

    s_all = jnp.concatenate(s1s + s2s, axis=0)
    sc_all, pos_all = _extract_topk(s_all, KNN, N_KEYS)

    half = 8 * TILE
    sc12 = jnp.concatenate([sc_all[:half], sc_all[half:]], axis=1)
    ix12 = jnp.concatenate(
        [pos_all[:half].astype(jnp.float32) * float(N_KEYS),
         pos_all[half:].astype(jnp.float32)], axis=1)
    cand_sc = jnp.dot(sc12, ab, precision=hi,
                      preferred_element_type=jnp.float32) + pad_mask
    cand_if = jnp.dot(ix12, ab, precision=hi,
                      preferred_element_type=jnp.float32)

    bsc, fidx_f = _extract_topk_payload_f(cand_sc, cand_if, KNN, N_CPAD)
    fidx = (fidx_f + 0.5).astype(jnp.int32)
    e = jnp.exp(bsc - bsc[:, :1])
    w = e / jnp.sum(e, axis=1, keepdims=True)
    w_ref[...] = w.reshape(8, TILE, KNN)
    idx_ref[...] = fidx.reshape(8, TILE, KNN)


def kernel(x, Wq_adv, bq_adv, keys_adv, values_adv, Wq_val, bq_val,
           keys_val, values_val):
    devs = jax.devices()
    if len(devs) >= 2:
        mesh = jax.sharding.Mesh(_np.array(devs[:2]), ("d",))
        P = jax.sharding.PartitionSpec
        rep = P()
        f = jax.shard_map(
            _kernel_impl, mesh=mesh,
            in_specs=(P("d"), rep, rep, rep, rep, rep, rep, rep, rep),
            out_specs=P("d"), check_vma=False)
        return f(x, Wq_adv, bq_adv, keys_adv, values_adv, Wq_val, bq_val,
                 keys_val, values_val)
    return _kernel_impl(x, Wq_adv, bq_adv, keys_adv, values_adv, Wq_val,
                        bq_val, keys_val, values_val)


def _kernel_impl(x, Wq_adv, bq_adv, keys_adv, values_adv, Wq_val, bq_val,
                 keys_val, values_val):
    bs = x.shape[0]
    wq = jnp.concatenate([Wq_adv, Wq_val], axis=1)
    bq = jnp.concatenate([bq_adv, bq_val])[None, :]
    k_adv = keys_adv.reshape(HEADS * 2, N_KEYS, HALF)
    k_val = keys_val.reshape(HEADS * 2, N_KEYS, HALF)
    ab = jnp.asarray(_np.concatenate([_A_MAT, _B_MAT], axis=0))
    pad_mask = jnp.asarray(_PAD_MASK)

    grid = (bs // TILE,)
    w_st, idx_st = pl.pallas_call(
        _select_kernel,
        grid=grid,
        in_specs=[
            pl.BlockSpec((TILE, HIDDEN), lambda i: (i, 0)),
            pl.BlockSpec((HIDDEN, 2 * HEADS * K_DIM), lambda i: (0, 0)),
            pl.BlockSpec((1, 2 * HEADS * K_DIM), lambda i: (0, 0)),
            pl.BlockSpec((HEADS * 2, N_KEYS, HALF), lambda i: (0, 0, 0)),
            pl.BlockSpec((HEADS * 2, N_KEYS, HALF), lambda i: (0, 0, 0)),
            pl.BlockSpec((2 * KNN, N_CPAD), lambda i: (0, 0)),
            pl.BlockSpec((1, N_CPAD), lambda i: (0, 0)),
        ],
        out_specs=[
            pl.BlockSpec((8, TILE, KNN), lambda i: (0, i, 0)),
            pl.BlockSpec((8, TILE, KNN), lambda i: (0, i, 0)),
        ],
        out_shape=[
            jax.ShapeDtypeStruct((8, bs, KNN), jnp.float32),
            jax.ShapeDtypeStruct((8, bs, KNN), jnp.int32),
        ],
    )(x, wq, bq, k_adv, k_val, ab, pad_mask)

    w_all = w_st.transpose(1, 0, 2).reshape(bs, 2 * HEADS * KNN)
    idx_all = idx_st.transpose(1, 0, 2).reshape(bs, 2 * HEADS * KNN)

    NPK = HEADS * KNN
    idx_adv = idx_all[:, :NPK].reshape(bs * NPK)
    idx_val = idx_all[:, NPK:].reshape(bs * NPK)
    vals_adv_pad = jnp.pad(values_adv, ((0, 0), (0, 128 - NUM_ACTIONS)))
    g_adv = _sc_gather(vals_adv_pad, idx_adv)
    g_val = _sc_gather_1d(values_val.reshape(N_VALUES), idx_val)
    g_val = g_val.reshape(bs * NPK, 1)
    iv_val = idx_all[:, NPK:]

    out = pl.pallas_call(
        _reduce_kernel,
        grid=(bs // RTILE,),
        in_specs=[
            pl.BlockSpec((RTILE, 2 * NPK), lambda i: (i, 0)),
            pl.BlockSpec((RTILE * NPK, 128), lambda i: (i, 0)),
            pl.BlockSpec((RTILE * NPK, 1), lambda i: (i, 0)),
            pl.BlockSpec((RTILE, NPK), lambda i: (i, 0)),
        ],
        out_specs=pl.BlockSpec((RTILE, NUM_ACTIONS), lambda i: (i, 0)),
        out_shape=jax.ShapeDtypeStruct((bs, NUM_ACTIONS), jnp.float32),
    )(w_all, g_adv, g_val, iv_val)
    return out


VPAD = 32
VQ = 64
RTILE = 64
GCHUNK = 512
GCHUNKV = 2048


def _sc_gather(table, idx):
    m = idx.shape[0]
    info = plsc.get_sparse_core_info()
    nw = info.num_cores * info.num_subcores
    per_w = m // nw
    n_chunks = per_w // GCHUNK
    mesh = plsc.VectorSubcoreMesh(core_axis_name="c", subcore_axis_name="s")

    @functools.partial(
        pl.kernel,
        out_type=jax.ShapeDtypeStruct((m, 128), table.dtype),
        mesh=mesh,
        scratch_types=[
            pltpu.VMEM((GCHUNK,), jnp.int32),
            pltpu.VMEM((GCHUNK, 128), jnp.float32),
            pltpu.SemaphoreType.DMA,
        ])
    def gather_kernel(tab_hbm, i_hbm, o_hbm, idx_v, rows_v, sem):
        wid = jax.lax.axis_index("s") * info.num_cores + jax.lax.axis_index("c")
        base = wid * per_w

        @pl.loop(0, n_chunks)
        def _(i):
            off = base + i * GCHUNK
            pltpu.sync_copy(i_hbm.at[pl.ds(off, GCHUNK)], idx_v)
            pltpu.async_copy(tab_hbm.at[idx_v], rows_v, sem).wait()
            pltpu.sync_copy(rows_v, o_hbm.at[pl.ds(off, GCHUNK)])

    return gather_kernel(table, idx)


def _sc_gather_spmem(table, idx):
    m = idx.shape[0]
    n16 = table.shape[0]
    info = plsc.get_sparse_core_info()
    nw = info.num_cores * info.num_subcores
    per_w = m // nw
    n_chunks = per_w // GCHUNK
    mesh = plsc.VectorSubcoreMesh(core_axis_name="c", subcore_axis_name="s")

    @functools.partial(
        pl.kernel,
        out_type=jax.ShapeDtypeStruct((m, VQ), jnp.float32),
        mesh=mesh,
        scratch_types=[
            pltpu.VMEM((GCHUNK,), jnp.int32),
            pltpu.VMEM((GCHUNK, VQ), jnp.float32),
            pltpu.VMEM_SHARED((n16, VQ), jnp.float32),
            pltpu.SemaphoreType.DMA,
        ])
    def gather_kernel(tab_hbm, i_hbm, o_hbm, idx_v, rows_v, tab_sh, sem):
        sid = jax.lax.axis_index("s")

        @pl.when(sid == 0)
        def _():
            pltpu.sync_copy(tab_hbm, tab_sh)

        plsc.subcore_barrier()
        wid = sid * info.num_cores + jax.lax.axis_index("c")
        base = wid * per_w

        @pl.loop(0, n_chunks)
        def _(i):
            off = base + i * GCHUNK
            pltpu.sync_copy(i_hbm.at[pl.ds(off, GCHUNK)], idx_v)
            pltpu.async_copy(tab_sh.at[idx_v], rows_v, sem).wait()
            pltpu.sync_copy(rows_v, o_hbm.at[pl.ds(off, GCHUNK)])

    return gather_kernel(table, idx)


def _sc_gather_1d(table, idx):
    m = idx.shape[0]
    info = plsc.get_sparse_core_info()
    nw = info.num_cores * info.num_subcores
    per_w = m // nw
    n_chunks = per_w // GCHUNKV
    mesh = plsc.VectorSubcoreMesh(core_axis_name="c", subcore_axis_name="s")

    @functools.partial(
        pl.kernel,
        out_type=jax.ShapeDtypeStruct((m,), jnp.float32),
        mesh=mesh,
        scratch_types=[
            pltpu.VMEM((GCHUNKV,), jnp.int32),
            pltpu.VMEM((GCHUNKV,), jnp.float32),
            pltpu.SemaphoreType.DMA,
        ])
    def gather_kernel(tab_hbm, i_hbm, o_hbm, idx_v, rows_v, sem):
        wid = jax.lax.axis_index("s") * info.num_cores + jax.lax.axis_index("c")
        base = wid * per_w

        @pl.loop(0, n_chunks)
        def _(i):
            off = base + i * GCHUNKV
            pltpu.sync_copy(i_hbm.at[pl.ds(off, GCHUNKV)], idx_v)
            pltpu.async_copy(tab_hbm.at[idx_v], rows_v, sem).wait()
            pltpu.sync_copy(rows_v, o_hbm.at[pl.ds(off, GCHUNKV)])

    return gather_kernel(table, idx)


def _reduce_kernel(w_ref, gadv_ref, gval_ref, iv_ref, out_ref):
    NPK = HEADS * KNN
    w_adv = w_ref[:, :NPK]
    w_val = w_ref[:, NPK:]
    g = gadv_ref[...].reshape(RTILE, NPK, 128)[:, :, :VPAD]
    adv = jnp.sum(g * w_adv[:, :, None], axis=1)
    gv = gval_ref[...].reshape(RTILE, NPK)
    val = jnp.sum(gv * w_val, axis=1, keepdims=True)
    advd = adv[:, :NUM_ACTIONS]
    mean = jnp.sum(advd, axis=1, keepdims=True) * (1.0 / NUM_ACTIONS)
    out_ref[...] = advd - mean + val


# speedup vs baseline: 12.0895x; 1.0682x over previous
import functools
import jax
import jax.numpy as jnp
from jax.experimental import pallas as pl
from jax.experimental.pallas import tpu as pltpu
from jax.experimental.pallas import tpu_sc as plsc

B = 16384
HIDDEN = 256
NUM_ACTIONS = 18
HEADS = 4
K_DIM = 256
HALF = K_DIM // 2
N_KEYS = 1024
KNN = 64
N_VALUES = N_KEYS * N_KEYS

TILE = 128

_PAIRS = [(a, b) for a in range(KNN) for b in range(KNN // (a + 1))]
N_CAND = len(_PAIRS)
N_CPAD = 384

import numpy as _np
_A_MAT = _np.zeros((KNN, N_CPAD), _np.float32)
_B_MAT = _np.zeros((KNN, N_CPAD), _np.float32)
_PAD_MASK = _np.full((1, N_CPAD), -1e30, _np.float32)
for _c, (_a, _b) in enumerate(_PAIRS):
    _A_MAT[_a, _c] = 1.0
    _B_MAT[_b, _c] = 1.0
    _PAD_MASK[0, _c] = 0.0

_NEG = float('-inf')


def _extract_topk(s, k, n_sentinel):
    R, N = s.shape
    iota = jax.lax.broadcasted_iota(jnp.int32, (R, N), 1)
    oiota = jax.lax.broadcasted_iota(jnp.int32, (R, k), 1)

    def body(j, carry):
        cur, outv, outp = carry
        m = jnp.max(cur, axis=1, keepdims=True)
        hit = cur == m
        p = jnp.min(jnp.where(hit, iota, n_sentinel), axis=1, keepdims=True)
        cur = jnp.where(iota == p, _NEG, cur)
        sel = oiota == j
        outv = jnp.where(sel, m, outv)
        outp = jnp.where(sel, p, outp)
        return cur, outv, outp

    outv = jnp.full((R, k), _NEG, jnp.float32)
    outp = jnp.zeros((R, k), jnp.int32)
    _, outv, outp = jax.lax.fori_loop(0, k, body, (s, outv, outp),
                                      unroll=8)
    return outv, outp


def _extract_topk_payload_f(s, payload, k, n_sentinel):
    R, N = s.shape
    iota = jax.lax.broadcasted_iota(jnp.int32, (R, N), 1)
    oiota = jax.lax.broadcasted_iota(jnp.int32, (R, k), 1)
    big = jnp.float32(3e9)

    def body(j, carry):
        cur, outv, outi = carry
        m = jnp.max(cur, axis=1, keepdims=True)
        hit = cur == m
        p = jnp.min(jnp.where(hit, iota, n_sentinel), axis=1, keepdims=True)
        at_p = iota == p
        pay = jnp.min(jnp.where(at_p, payload, big), axis=1, keepdims=True)
        cur = jnp.where(at_p, _NEG, cur)
        sel = oiota == j
        outv = jnp.where(sel, m, outv)
        outi = jnp.where(sel, pay, outi)
        return cur, outv, outi

    outv = jnp.full((R, k), _NEG, jnp.float32)
    outi = jnp.zeros((R, k), jnp.float32)
    _, outv, outi = jax.lax.fori_loop(0, k, body, (s, outv, outi),
                                      unroll=8)
    return outv, outi


def _select_kernel(x_ref, wq_ref, bq_ref, k_adv_ref, k_val_ref,
                   ab_ref, mask_ref, w_ref, idx_ref):
    x = x_ref[...]
    q = jnp.dot(x, wq_ref[...], preferred_element_type=jnp.float32)
    q = q + bq_ref[0, :]
    ab = ab_ref[...]
    pad_mask = mask_ref[0, :]
    hi = jax.lax.Precision.HIGHEST

    s1s, s2s = [], []
    for pkm in range(2):
        k_ref = k_adv_ref if pkm == 0 else k_val_ref
        for h in range(HEADS):
            base = pkm * HEADS * K_DIM + h * K_DIM
            q1 = q[:, base:base + HALF]
            q2 = q[:, base + HALF:base + K_DIM]
            s1s.append(jax.lax.dot_general(
                q1, k_ref[2 * h], (((1,), (1,)), ((), ())),
                preferred_element_type=jnp.float32))
            s2s.append(jax.lax.dot_general(
                q2, k_ref[2 * h + 1], (((1,), (1,)), ((), ())),
                preferred_element_type=jnp.float32))

    s_all = jnp.concatenate(s1s + s2s, axis=0)
    sc_all, pos_all = _extract_topk(s_all, KNN, N_KEYS)

    half = 8 * TILE
    sc12 = jnp.concatenate([sc_all[:half], sc_all[half:]], axis=1)
    ix12 = jnp.concatenate(
        [pos_all[:half].astype(jnp.float32) * float(N_KEYS),
         pos_all[half:].astype(jnp.float32)], axis=1)
    cand_sc = jnp.dot(sc12, ab, precision=hi,
                      preferred_element_type=jnp.float32) + pad_mask
    cand_if = jnp.dot(ix12, ab, precision=hi,
                      preferred_element_type=jnp.float32)

    bsc, fidx_f = _extract_topk_payload_f(cand_sc, cand_if, KNN, N_CPAD)
    fidx = (fidx_f + 0.5).astype(jnp.int32)
    e = jnp.exp(bsc - bsc[:, :1])
    w = e / jnp.sum(e, axis=1, keepdims=True)
    w_ref[...] = w.reshape(8, TILE, KNN)
    idx_ref[...] = fidx.reshape(8, TILE, KNN)


def kernel(x, Wq_adv, bq_adv, keys_adv, values_adv, Wq_val, bq_val,
           keys_val, values_val):
    devs = jax.devices()
    if len(devs) >= 2:
        mesh = jax.sharding.Mesh(_np.array(devs[:2]), ("d",))
        P = jax.sharding.PartitionSpec
        rep = P()
        f = jax.shard_map(
            _kernel_impl, mesh=mesh,
            in_specs=(P("d"), rep, rep, rep, rep, rep, rep, rep, rep),
            out_specs=P("d"), check_vma=False)
        return f(x, Wq_adv, bq_adv, keys_adv, values_adv, Wq_val, bq_val,
                 keys_val, values_val)
    return _kernel_impl(x, Wq_adv, bq_adv, keys_adv, values_adv, Wq_val,
                        bq_val, keys_val, values_val)


def _kernel_impl(x, Wq_adv, bq_adv, keys_adv, values_adv, Wq_val, bq_val,
                 keys_val, values_val):
    bs = x.shape[0]
    wq = jnp.concatenate([Wq_adv, Wq_val], axis=1)
    bq = jnp.concatenate([bq_adv, bq_val])[None, :]
    k_adv = keys_adv.reshape(HEADS * 2, N_KEYS, HALF)
    k_val = keys_val.reshape(HEADS * 2, N_KEYS, HALF)
    ab = jnp.asarray(_np.concatenate([_A_MAT, _B_MAT], axis=0))
    pad_mask = jnp.asarray(_PAD_MASK)

    grid = (bs // TILE,)
    w_st, idx_st = pl.pallas_call(
        _select_kernel,
        grid=grid,
        in_specs=[
            pl.BlockSpec((TILE, HIDDEN), lambda i: (i, 0)),
            pl.BlockSpec((HIDDEN, 2 * HEADS * K_DIM), lambda i: (0, 0)),
            pl.BlockSpec((1, 2 * HEADS * K_DIM), lambda i: (0, 0)),
            pl.BlockSpec((HEADS * 2, N_KEYS, HALF), lambda i: (0, 0, 0)),
            pl.BlockSpec((HEADS * 2, N_KEYS, HALF), lambda i: (0, 0, 0)),
            pl.BlockSpec((2 * KNN, N_CPAD), lambda i: (0, 0)),
            pl.BlockSpec((1, N_CPAD), lambda i: (0, 0)),
        ],
        out_specs=[
            pl.BlockSpec((8, TILE, KNN), lambda i: (0, i, 0)),
            pl.BlockSpec((8, TILE, KNN), lambda i: (0, i, 0)),
        ],
        out_shape=[
            jax.ShapeDtypeStruct((8, bs, KNN), jnp.float32),
            jax.ShapeDtypeStruct((8, bs, KNN), jnp.int32),
        ],
    )(x, wq, bq, k_adv, k_val, ab, pad_mask)

    w_all = w_st.transpose(1, 0, 2).reshape(bs, 2 * HEADS * KNN)
    idx_all = idx_st.transpose(1, 0, 2).reshape(bs, 2 * HEADS * KNN)

    NPK = HEADS * KNN
    idx_adv = idx_all[:, :NPK].reshape(bs * NPK)
    idx_val = idx_all[:, NPK:].reshape(bs * NPK)
    vals_adv_pad = jnp.pad(values_adv, ((0, 0), (0, 128 - NUM_ACTIONS)))
    g_adv = _sc_gather(vals_adv_pad, idx_adv)
    g_val = _sc_gather_1d(values_val.reshape(N_VALUES), idx_val)
    g_val = g_val.reshape(bs * NPK, 1)
    iv_val = idx_all[:, NPK:]

    out = pl.pallas_call(
        _reduce_kernel,
        grid=(bs // RTILE,),
        in_specs=[
            pl.BlockSpec((RTILE, 2 * NPK), lambda i: (i, 0)),
            pl.BlockSpec((RTILE * NPK, 128), lambda i: (i, 0)),
            pl.BlockSpec((RTILE * NPK, 1), lambda i: (i, 0)),
            pl.BlockSpec((RTILE, NPK), lambda i: (i, 0)),
        ],
        out_specs=pl.BlockSpec((RTILE, NUM_ACTIONS), lambda i: (i, 0)),
        out_shape=jax.ShapeDtypeStruct((bs, NUM_ACTIONS), jnp.float32),
    )(w_all, g_adv, g_val, iv_val)
    return out


VPAD = 32
VQ = 64
RTILE = 64
GCHUNK = 512
GCHUNKV = 2048


def _sc_gather(table, idx):
    m = idx.shape[0]
    info = plsc.get_sparse_core_info()
    nw = info.num_cores * info.num_subcores
    per_w = m // nw
    n_chunks = per_w // GCHUNK
    mesh = plsc.VectorSubcoreMesh(core_axis_name="c", subcore_axis_name="s")

    @functools.partial(
        pl.kernel,
        out_type=jax.ShapeDtypeStruct((m, 128), table.dtype),
        mesh=mesh,
        scratch_types=[
            pltpu.VMEM((GCHUNK,), jnp.int32),
            pltpu.VMEM((GCHUNK, 128), jnp.float32),
            pltpu.SemaphoreType.DMA,
        ])
    def gather_kernel(tab_hbm, i_hbm, o_hbm, idx_v, rows_v, sem):
        wid = jax.lax.axis_index("s") * info.num_cores + jax.lax.axis_index("c")
        base = wid * per_w

        @pl.loop(0, n_chunks)
        def _(i):
            off = base + i * GCHUNK
            pltpu.sync_copy(i_hbm.at[pl.ds(off, GCHUNK)], idx_v)
            pltpu.async_copy(tab_hbm.at[idx_v], rows_v, sem).wait()
            pltpu.sync_copy(rows_v, o_hbm.at[pl.ds(off, GCHUNK)])

    return gather_kernel(table, idx)


def _sc_gather_spmem(table, idx):
    m = idx.shape[0]
    n16 = table.shape[0]
    info = plsc.get_sparse_core_info()
    nw = info.num_cores * info.num_subcores
    per_w = m // nw
    n_chunks = per_w // GCHUNK
    mesh = plsc.VectorSubcoreMesh(core_axis_name="c", subcore_axis_name="s")

    @functools.partial(
        pl.kernel,
        out_type=jax.ShapeDtypeStruct((m, VQ), jnp.float32),
        mesh=mesh,
        scratch_types=[
            pltpu.VMEM((GCHUNK,), jnp.int32),
            pltpu.VMEM((GCHUNK, VQ), jnp.float32),
            pltpu.VMEM_SHARED((n16, VQ), jnp.float32),
            pltpu.SemaphoreType.DMA,
        ])
    def gather_kernel(tab_hbm, i_hbm, o_hbm, idx_v, rows_v, tab_sh, sem):
        sid = jax.lax.axis_index("s")

        @pl.when(sid == 0)
        def _():
            pltpu.sync_copy(tab_hbm, tab_sh)

        plsc.subcore_barrier()
        wid = sid * info.num_cores + jax.lax.axis_index("c")
        base = wid * per_w

        @pl.loop(0, n_chunks)
        def _(i):
            off = base + i * GCHUNK
            pltpu.sync_copy(i_hbm.at[pl.ds(off, GCHUNK)], idx_v)
            pltpu.async_copy(tab_sh.at[idx_v], rows_v, sem).wait()
            pltpu.sync_copy(rows_v, o_hbm.at[pl.ds(off, GCHUNK)])

    return gather_kernel(table, idx)


def _sc_gather_1d(table, idx):
    m = idx.shape[0]
    info = plsc.get_sparse_core_info()
    nw = info.num_cores * info.num_subcores
    per_w = m // nw
    n_chunks = per_w // GCHUNKV
    mesh = plsc.VectorSubcoreMesh(core_axis_name="c", subcore_axis_name="s")

    @functools.partial(
        pl.kernel,
        out_type=jax.ShapeDtypeStruct((m,), jnp.float32),
        mesh=mesh,
        scratch_types=[
            pltpu.VMEM((GCHUNKV,), jnp.int32),
            pltpu.VMEM((GCHUNKV,), jnp.float32),
            pltpu.SemaphoreType.DMA,
        ])
    def gather_kernel(tab_hbm, i_hbm, o_hbm, idx_v, rows_v, sem):
        wid = jax.lax.axis_index("s") * info.num_cores + jax.lax.axis_index("c")
        base = wid * per_w

        @pl.loop(0, n_chunks)
        def _(i):
            off = base + i * GCHUNKV
            pltpu.sync_copy(i_hbm.at[pl.ds(off, GCHUNKV)], idx_v)
            pltpu.async_copy(tab_hbm.at[idx_v], rows_v, sem).wait()
            pltpu.sync_copy(rows_v, o_hbm.at[pl.ds(off, GCHUNKV)])

    return gather_kernel(table, idx)


def _reduce_kernel(w_ref, gadv_ref, gval_ref, iv_ref, out_ref):
    NPK = HEADS * KNN
    w_adv = w_ref[:, :NPK]
    w_val = w_ref[:, NPK:]
    g = gadv_ref[...].reshape(RTILE, NPK, 128)[:, :, :VPAD]
    adv = jnp.sum(g * w_adv[:, :, None], axis=1)
    gv = gval_ref[...].reshape(RTILE, NPK)
    val = jnp.sum(gv * w_val, axis=1, keepdims=True)
    advd = adv[:, :NUM_ACTIONS]
    mean = jnp.sum(advd, axis=1, keepdims=True) * (1.0 / NUM_ACTIONS)
    out_ref[...] = advd - mean + val
